# Initial kernel scaffold; baseline (speedup 1.0000x reference)
import jax
import jax.numpy as jnp
from jax.experimental import pallas as pl


def kernel(x, edge_index):
    raise NotImplementedError("write your pallas kernel here")


# speedup vs baseline: 5.8004x; 5.8004x over previous
import functools

import jax
import jax.numpy as jnp
from jax import lax
from jax.experimental import pallas as pl
from jax.experimental.pallas import tpu as pltpu
from jax.experimental.pallas import tpu_sc as plsc

N = 10000
NP = 10240
D = 128
E = 320000
DH = D // 2
NS = 16
C = 80
EPT = E // NS
CHUNKS = EPT // C
ROWS_PT = NP // NS
RB = 128
NB = ROWS_PT // RB
RPC = EPT // C


def _zero_2d(ref, nrows, ncols):
    zeros = jnp.zeros((16,), jnp.float32)

    def body(r, _):
        for k in range(ncols // 16):
            ref[r, pl.ds(k * 16, 16)] = zeros
        return 0

    lax.fori_loop(0, nrows, body, 0)


def _edge_loop(table_hbm, src_v, dst_v, rows_v, sem, acc_sp, cnt_sp, ones_v,
               with_count):
    def body(i, _):
        pltpu.async_copy(table_hbm.at[src_v.at[i]], rows_v, sem).wait()
        pltpu.sync_copy(rows_v, acc_sp.at[dst_v.at[i]], add=True)
        if with_count:
            pltpu.sync_copy(ones_v, cnt_sp.at[dst_v.at[i]], add=True)
        return 0

    lax.fori_loop(0, CHUNKS, body, 0)


def _finalize(s, c, acc_sp, cnt_sp, accbuf_v, cntbuf_v, recip_v, dst_hbm,
              first_layer):
    for b in range(NB):
        base = s * ROWS_PT + b * RB
        pltpu.sync_copy(acc_sp.at[pl.ds(base, RB)], accbuf_v)
        if first_layer:
            pltpu.sync_copy(cnt_sp.at[pl.ds(base, RB)], cntbuf_v)

        def body(r, _):
            if first_layer:
                q = 1.0 / jnp.maximum(cntbuf_v[r, pl.ds(0, 16)], 1.0)
                recip_v[b * RB + r, pl.ds(0, 16)] = q
            else:
                q = recip_v[b * RB + r, pl.ds(0, 16)]
            for k in range(DH // 16):
                accbuf_v[r, pl.ds(k * 16, 16)] = (
                    accbuf_v[r, pl.ds(k * 16, 16)] * q)
            return 0

        lax.fori_loop(0, RB, body, 0)
        pltpu.sync_copy(accbuf_v, dst_hbm.at[pl.ds(c * NP + base, RB)])


def _gcn_body(x_hbm, src_hbm, dst_hbm, h1_hbm, out_hbm,
              src_v, dst_v, rows_v, ones_v, accbuf_v, cntbuf_v, recip_v,
              acc_sp, cnt_sp, sem):
    s = lax.axis_index("s")
    c = lax.axis_index("c")

    pltpu.sync_copy(src_hbm.at[s], src_v)
    pltpu.sync_copy(dst_hbm.at[s], dst_v)
    off = (c * NP).astype(jnp.int32) + jnp.zeros((16,), jnp.int32)

    def shift(r, _):
        for k in range(C // 16):
            src_v[r, pl.ds(k * 16, 16)] = src_v[r, pl.ds(k * 16, 16)] + off
        return 0

    lax.fori_loop(0, RPC, shift, 0)

    one16 = jnp.ones((16,), jnp.float32)

    def fill_ones(r, _):
        ones_v[r, pl.ds(0, 16)] = one16
        return 0

    lax.fori_loop(0, C, fill_ones, 0)

    _zero_2d(accbuf_v, RB, DH)
    _zero_2d(cntbuf_v, RB, 16)
    for b in range(NB):
        pltpu.sync_copy(accbuf_v, acc_sp.at[pl.ds(s * ROWS_PT + b * RB, RB)])
        pltpu.sync_copy(cntbuf_v, cnt_sp.at[pl.ds(s * ROWS_PT + b * RB, RB)])
    plsc.subcore_barrier()

    _edge_loop(x_hbm, src_v, dst_v, rows_v, sem, acc_sp, cnt_sp, ones_v, True)
    plsc.subcore_barrier()

    _finalize(s, c, acc_sp, cnt_sp, accbuf_v, cntbuf_v, recip_v, h1_hbm, True)
    _zero_2d(accbuf_v, RB, DH)
    for b in range(NB):
        pltpu.sync_copy(accbuf_v, acc_sp.at[pl.ds(s * ROWS_PT + b * RB, RB)])
    plsc.subcore_barrier()

    _edge_loop(h1_hbm, src_v, dst_v, rows_v, sem, acc_sp, cnt_sp, ones_v,
               False)
    plsc.subcore_barrier()

    _finalize(s, c, acc_sp, cnt_sp, accbuf_v, cntbuf_v, recip_v, out_hbm,
              False)


@jax.jit
def _gcn(x2, src2, dst2):
    f32 = jnp.float32
    kern = pl.kernel(
        _gcn_body,
        mesh=plsc.VectorSubcoreMesh(core_axis_name="c", subcore_axis_name="s"),
        out_type=[
            jax.ShapeDtypeStruct((2 * NP, DH), f32),
            jax.ShapeDtypeStruct((2 * NP, DH), f32),
        ],
        scratch_types=[
            pltpu.VMEM((RPC, C), jnp.int32),
            pltpu.VMEM((RPC, C), jnp.int32),
            pltpu.VMEM((C, DH), f32),
            pltpu.VMEM((C, 16), f32),
            pltpu.VMEM((RB, DH), f32),
            pltpu.VMEM((RB, 16), f32),
            pltpu.VMEM((ROWS_PT, 16), f32),
            pltpu.VMEM_SHARED((NP, DH), f32),
            pltpu.VMEM_SHARED((NP, 16), f32),
            pltpu.SemaphoreType.DMA,
        ],
        compiler_params=pltpu.CompilerParams(use_tc_tiling_on_sc=False),
    )
    _, out2 = kern(x2, src2, dst2)
    return out2


def kernel(x, edge_index):
    x2 = jnp.zeros((2 * NP, DH), jnp.float32)
    x2 = x2.at[:N].set(x[:, :DH]).at[NP:NP + N].set(x[:, DH:])
    src2 = edge_index[0].reshape(NS, RPC, C)
    dst2 = edge_index[1].reshape(NS, RPC, C)
    x2, src2, dst2 = lax.optimization_barrier((x2, src2, dst2))
    out2 = _gcn(x2, src2, dst2)
    out2 = lax.optimization_barrier(out2)
    return jnp.concatenate([out2[:N], out2[NP:NP + N]], axis=1)


# speedup vs baseline: 11.7365x; 2.0234x over previous
import jax
import jax.numpy as jnp
from jax import lax
from jax.experimental import pallas as pl
from jax.experimental.pallas import tpu as pltpu
from jax.experimental.pallas import tpu_sc as plsc

N = 10000
NP = 10240
D = 128
E = 320000
DH = D // 2
NS = 16
C = 100
EPT = E // NS
CHUNKS = EPT // C
NBUF = 4
ROWS_PT = NP // NS
RB = 128
NB = ROWS_PT // RB


def _zero_2d(ref, nrows, ncols):
    zeros = jnp.zeros((16,), jnp.float32)

    def body(r, _):
        for k in range(ncols // 16):
            ref[r, pl.ds(k * 16, 16)] = zeros
        return 0

    lax.fori_loop(0, nrows, body, 0)


def _edge_loop(table_hbm, src_v, dst_v, rows, ones_v, acc_sp, cnt_sp,
               sem_g, sem_s, sem_cnt, with_count):
    def tab(i):
        return table_hbm.at[src_v.at[i]]

    pltpu.async_copy(tab(0), rows[0], sem_g[0])
    pltpu.async_copy(tab(1), rows[1], sem_g[1])

    def body(g, _):
        for b in range(NBUF):
            i = g * NBUF + b
            bn = (b + 2) % NBUF

            pltpu.make_async_copy(tab(i), rows[b], sem_g[b]).wait()

            if with_count:
                @pl.when(i >= NBUF)
                def _():
                    pltpu.make_async_copy(
                        ones_v, cnt_sp.at[dst_v.at[0]], sem_cnt[b]).wait()

                pltpu.async_copy(
                    ones_v, cnt_sp.at[dst_v.at[i]], sem_cnt[b], add=True)
            pltpu.async_copy(
                rows[b], acc_sp.at[dst_v.at[i]], sem_s[b], add=True)

            @pl.when(i >= 2)
            def _():
                pltpu.make_async_copy(
                    rows[bn], acc_sp.at[dst_v.at[0]], sem_s[bn]).wait()

            @pl.when(i + 2 < CHUNKS)
            def _():
                pltpu.async_copy(tab(i + 2), rows[bn], sem_g[bn])
        return 0

    lax.fori_loop(0, CHUNKS // NBUF, body, 0)

    for i in (CHUNKS - 2, CHUNKS - 1):
        pltpu.make_async_copy(
            rows[i % NBUF], acc_sp.at[dst_v.at[0]], sem_s[i % NBUF]).wait()
    if with_count:
        for b in range(NBUF):
            pltpu.make_async_copy(
                ones_v, cnt_sp.at[dst_v.at[0]], sem_cnt[b]).wait()


def _finalize(s, c, acc_sp, cnt_sp, accbuf_v, cntbuf_v, dst_hbm, col_slice):
    for b in range(NB):
        base = s * ROWS_PT + b * RB
        pltpu.sync_copy(acc_sp.at[pl.ds(base, RB)], accbuf_v)
        pltpu.sync_copy(cnt_sp.at[pl.ds(base, RB)], cntbuf_v)

        def body(r, _):
            q = 1.0 / jnp.maximum(cntbuf_v[r, pl.ds(0, 16)], 1.0)
            for k in range(DH // 16):
                accbuf_v[r, pl.ds(k * 16, 16)] = (
                    accbuf_v[r, pl.ds(k * 16, 16)] * q)
            return 0

        lax.fori_loop(0, RB, body, 0)
        if col_slice:
            pltpu.sync_copy(
                accbuf_v, dst_hbm.at[pl.ds(base, RB), pl.ds(c * DH, DH)])
        else:
            pltpu.sync_copy(accbuf_v, dst_hbm.at[pl.ds(c * NP + base, RB)])


def _gcn_body(x_hbm, src_hbm, dst_hbm, h1_hbm, out_hbm,
              src_v, dst_v, rows0_v, rows1_v, rows2_v, rows3_v, ones_v,
              accbuf_v, cntbuf_v, acc_sp, cnt_sp,
              sg0, sg1, sg2, sg3, ss0, ss1, ss2, ss3, sc0, sc1, sc2, sc3):
    s = lax.axis_index("s")
    c = lax.axis_index("c")
    rows = (rows0_v, rows1_v, rows2_v, rows3_v)
    sem_g = (sg0, sg1, sg2, sg3)
    sem_s = (ss0, ss1, ss2, ss3)
    sem_cnt = (sc0, sc1, sc2, sc3)

    pltpu.sync_copy(src_hbm.at[c, s], src_v)
    pltpu.sync_copy(dst_hbm.at[s], dst_v)

    one16 = jnp.ones((16,), jnp.float32)

    def fill_ones(r, _):
        ones_v[r, pl.ds(0, 16)] = one16
        return 0

    lax.fori_loop(0, C, fill_ones, 0)

    _zero_2d(accbuf_v, RB, DH)
    _zero_2d(cntbuf_v, RB, 16)
    for b in range(NB):
        pltpu.sync_copy(accbuf_v, acc_sp.at[pl.ds(s * ROWS_PT + b * RB, RB)])
        pltpu.sync_copy(cntbuf_v, cnt_sp.at[pl.ds(s * ROWS_PT + b * RB, RB)])
    plsc.subcore_barrier()

    _edge_loop(x_hbm, src_v, dst_v, rows, ones_v, acc_sp, cnt_sp,
               sem_g, sem_s, sem_cnt, True)
    plsc.subcore_barrier()

    _finalize(s, c, acc_sp, cnt_sp, accbuf_v, cntbuf_v, h1_hbm, False)
    _zero_2d(accbuf_v, RB, DH)
    for b in range(NB):
        pltpu.sync_copy(accbuf_v, acc_sp.at[pl.ds(s * ROWS_PT + b * RB, RB)])
    plsc.subcore_barrier()

    _edge_loop(h1_hbm, src_v, dst_v, rows, ones_v, acc_sp, cnt_sp,
               sem_g, sem_s, sem_cnt, False)
    plsc.subcore_barrier()

    _finalize(s, c, acc_sp, cnt_sp, accbuf_v, cntbuf_v, out_hbm, True)


@jax.jit
def _gcn(x2, src4, dst3):
    f32 = jnp.float32
    kern = pl.kernel(
        _gcn_body,
        mesh=plsc.VectorSubcoreMesh(core_axis_name="c", subcore_axis_name="s"),
        out_type=[
            jax.ShapeDtypeStruct((2 * NP, DH), f32),
            jax.ShapeDtypeStruct((NP, D), f32),
        ],
        scratch_types=[
            pltpu.VMEM((CHUNKS, C), jnp.int32),
            pltpu.VMEM((CHUNKS, C), jnp.int32),
            pltpu.VMEM((C, DH), f32),
            pltpu.VMEM((C, DH), f32),
            pltpu.VMEM((C, DH), f32),
            pltpu.VMEM((C, DH), f32),
            pltpu.VMEM((C, 16), f32),
            pltpu.VMEM((RB, DH), f32),
            pltpu.VMEM((RB, 16), f32),
            pltpu.VMEM_SHARED((NP, DH), f32),
            pltpu.VMEM_SHARED((NP, 16), f32),
        ] + [pltpu.SemaphoreType.DMA] * 12,
        compiler_params=pltpu.CompilerParams(use_tc_tiling_on_sc=False),
    )
    _, out2 = kern(x2, src4, dst3)
    return out2


def kernel(x, edge_index):
    x2 = jnp.zeros((2 * NP, DH), jnp.float32)
    x2 = x2.at[:N].set(x[:, :DH]).at[NP:NP + N].set(x[:, DH:])
    src3 = edge_index[0].reshape(NS, CHUNKS, C)
    src4 = jnp.stack([src3, src3 + NP])
    dst3 = edge_index[1].reshape(NS, CHUNKS, C)
    x2, src4, dst3 = lax.optimization_barrier((x2, src4, dst3))
    out2 = _gcn(x2, src4, dst3)
    out2 = lax.optimization_barrier(out2)
    return out2[:N]


# speedup vs baseline: 11.7393x; 1.0002x over previous
import jax
import jax.numpy as jnp
from jax import lax
from jax.experimental import pallas as pl
from jax.experimental.pallas import tpu as pltpu
from jax.experimental.pallas import tpu_sc as plsc

N = 10000
NP = 10240
D = 128
E = 320000
DH = D // 2
NS = 16
C = 125
EPT = E // NS
CHUNKS = EPT // C
CW = 16
NBUF = 4
ROWS_PT = NP // NS
RB = 128
NB = ROWS_PT // RB


def _zero_2d(ref, nrows, ncols):
    zeros = jnp.zeros((16,), jnp.float32)

    def body(r, _):
        for k in range(ncols // 16):
            ref[r, pl.ds(k * 16, 16)] = zeros
        return 0

    lax.fori_loop(0, nrows, body, 0)


def _edge_loop(table_hbm, src_v, dst_hbm, s, dstb, rows, ones_v, acc_sp,
               cnt_sp, sem_g, sem_di, sem_s, sem_cnt, with_count):
    def tab(i):
        return table_hbm.at[src_v.at[i]]

    def dst_load(i, j):
        pltpu.async_copy(dst_hbm.at[s, i], dstb[j], sem_di[j])

    pltpu.async_copy(tab(0), rows[0], sem_g[0])
    pltpu.async_copy(tab(1), rows[1], sem_g[1])
    dst_load(0, 0)
    dst_load(1, 1)

    def body(g, _):
        for b in range(NBUF):
            i = g * NBUF + b
            bn = (b + 2) % NBUF

            pltpu.make_async_copy(tab(i), rows[b], sem_g[b]).wait()
            pltpu.make_async_copy(
                dst_hbm.at[s, 0], dstb[b], sem_di[b]).wait()

            if with_count:
                pltpu.async_copy(
                    ones_v, cnt_sp.at[dstb[b]], sem_cnt[b], add=True)
            pltpu.async_copy(
                rows[b], acc_sp.at[dstb[b]], sem_s[b], add=True)

            @pl.when(i >= 2)
            def _():
                pltpu.make_async_copy(
                    rows[bn], acc_sp.at[dstb[bn]], sem_s[bn]).wait()
                if with_count:
                    pltpu.make_async_copy(
                        ones_v, cnt_sp.at[dstb[bn]], sem_cnt[bn]).wait()

            @pl.when(i + 2 < CHUNKS)
            def _():
                pltpu.async_copy(tab(i + 2), rows[bn], sem_g[bn])
                dst_load(i + 2, bn)
        return 0

    lax.fori_loop(0, CHUNKS // NBUF, body, 0)

    for i in (CHUNKS - 2, CHUNKS - 1):
        b = i % NBUF
        pltpu.make_async_copy(
            rows[b], acc_sp.at[dstb[b]], sem_s[b]).wait()
        if with_count:
            pltpu.make_async_copy(
                ones_v, cnt_sp.at[dstb[b]], sem_cnt[b]).wait()


def _finalize(s, c, acc_sp, cnt_sp, accbuf_v, cntbuf_v, dst_hbm, col_slice):
    for b in range(NB):
        base = s * ROWS_PT + b * RB
        pltpu.sync_copy(acc_sp.at[pl.ds(base, RB)], accbuf_v)
        pltpu.sync_copy(cnt_sp.at[pl.ds(base, RB)], cntbuf_v)

        def body(r, _):
            q = 1.0 / jnp.maximum(cntbuf_v[r, pl.ds(0, 16)], 1.0)
            for k in range(DH // 16):
                accbuf_v[r, pl.ds(k * 16, 16)] = (
                    accbuf_v[r, pl.ds(k * 16, 16)] * q)
            return 0

        lax.fori_loop(0, RB, body, 0)
        if col_slice:
            pltpu.sync_copy(
                accbuf_v, dst_hbm.at[pl.ds(base, RB), pl.ds(c * DH, DH)])
        else:
            pltpu.sync_copy(accbuf_v, dst_hbm.at[pl.ds(c * NP + base, RB)])


def _gcn_body(x_hbm, src_hbm, dst_hbm, ones_hbm, zeros_hbm, h1_hbm, out_hbm,
              src_v, dstb0, dstb1, dstb2, dstb3,
              rows0_v, rows1_v, rows2_v, rows3_v, ones_v,
              accbuf_v, cntbuf_v, acc_sp, cnt_sp, *sems):
    s = lax.axis_index("s")
    c = lax.axis_index("c")
    rows = (rows0_v, rows1_v, rows2_v, rows3_v)
    dstb = (dstb0, dstb1, dstb2, dstb3)
    sem_g = sems[0:4]
    sem_di = sems[4:8]
    sem_s = sems[8:12]
    sem_cnt = sems[12:16]

    pltpu.sync_copy(src_hbm.at[c, s], src_v)

    pltpu.sync_copy(ones_hbm, ones_v)
    pltpu.sync_copy(zeros_hbm, cntbuf_v)

    _zero_2d(accbuf_v, RB, DH)
    for b in range(NB):
        pltpu.sync_copy(accbuf_v, acc_sp.at[pl.ds(s * ROWS_PT + b * RB, RB)])
        pltpu.sync_copy(cntbuf_v, cnt_sp.at[pl.ds(s * ROWS_PT + b * RB, RB)])
    plsc.subcore_barrier()

    _edge_loop(x_hbm, src_v, dst_hbm, s, dstb, rows, ones_v, acc_sp,
               cnt_sp, sem_g, sem_di, sem_s, sem_cnt, True)
    plsc.subcore_barrier()

    _finalize(s, c, acc_sp, cnt_sp, accbuf_v, cntbuf_v, h1_hbm, False)
    _zero_2d(accbuf_v, RB, DH)
    for b in range(NB):
        pltpu.sync_copy(accbuf_v, acc_sp.at[pl.ds(s * ROWS_PT + b * RB, RB)])
    plsc.subcore_barrier()

    _edge_loop(h1_hbm, src_v, dst_hbm, s, dstb, rows, ones_v, acc_sp,
               cnt_sp, sem_g, sem_di, sem_s, sem_cnt, False)
    plsc.subcore_barrier()

    _finalize(s, c, acc_sp, cnt_sp, accbuf_v, cntbuf_v, out_hbm, True)


@jax.jit
def _gcn(x2, src4, dst3, ones8, zeros8):
    f32 = jnp.float32
    kern = pl.kernel(
        _gcn_body,
        mesh=plsc.VectorSubcoreMesh(core_axis_name="c", subcore_axis_name="s"),
        out_type=[
            jax.ShapeDtypeStruct((2 * NP, DH), f32),
            jax.ShapeDtypeStruct((NP, D), f32),
        ],
        scratch_types=[
            pltpu.VMEM((CHUNKS, C), jnp.int32),
            pltpu.VMEM((C,), jnp.int32),
            pltpu.VMEM((C,), jnp.int32),
            pltpu.VMEM((C,), jnp.int32),
            pltpu.VMEM((C,), jnp.int32),
            pltpu.VMEM((C, DH), f32),
            pltpu.VMEM((C, DH), f32),
            pltpu.VMEM((C, DH), f32),
            pltpu.VMEM((C, DH), f32),
            pltpu.VMEM((C, CW), f32),
            pltpu.VMEM((RB, DH), f32),
            pltpu.VMEM((RB, CW), f32),
            pltpu.VMEM_SHARED((NP, DH), f32),
            pltpu.VMEM_SHARED((NP, CW), f32),
        ] + [pltpu.SemaphoreType.DMA] * 16,
        compiler_params=pltpu.CompilerParams(use_tc_tiling_on_sc=False),
    )
    _, out2 = kern(x2, src4, dst3, ones8, zeros8)
    return out2


def kernel(x, edge_index):
    gap = jnp.zeros((NP - N, DH), jnp.float32)
    x2 = jnp.concatenate([x[:, :DH], gap, x[:, DH:]])
    src3 = edge_index[0].reshape(NS, CHUNKS, C)
    src4 = jnp.stack([src3, src3 + NP])
    dst3 = edge_index[1].reshape(NS, CHUNKS, C)
    ones8 = jnp.ones((C, CW), jnp.float32)
    zeros8 = jnp.zeros((RB, CW), jnp.float32)
    x2, src4, dst3 = lax.optimization_barrier((x2, src4, dst3))
    out2 = _gcn(x2, src4, dst3, ones8, zeros8)
    out2 = lax.optimization_barrier(out2)
    return out2[:N]


# speedup vs baseline: 11.8920x; 1.0130x over previous
import jax
import jax.numpy as jnp
from jax import lax
from jax.experimental import pallas as pl
from jax.experimental.pallas import tpu as pltpu
from jax.experimental.pallas import tpu_sc as plsc

N = 10000
NP = 10240
D = 128
E = 320000
DH = D // 2
NS = 16
C = 125
EPT = E // NS
CHUNKS = EPT // C
CW = 16
NBUF = 4
ROWS_PT = NP // NS
RB = 128
NB = ROWS_PT // RB


def _zero_2d(ref, nrows, ncols):
    zeros = jnp.zeros((16,), jnp.float32)

    def body(r, _):
        for k in range(ncols // 16):
            ref[r, pl.ds(k * 16, 16)] = zeros
        return 0

    lax.fori_loop(0, nrows, body, 0)


def _edge_loop(table_hbm, src_v, dst_hbm, s, dstb, rows, ones_v, acc_sp,
               cnt_sp, sem_g, sem_di, sem_s, sem_cnt, with_count):
    def tab(i):
        return table_hbm.at[src_v.at[i]]

    def dst_load(i, j):
        pltpu.async_copy(dst_hbm.at[s, i], dstb[j], sem_di[j])

    pltpu.async_copy(tab(0), rows[0], sem_g[0])
    pltpu.async_copy(tab(1), rows[1], sem_g[1])
    dst_load(0, 0)
    dst_load(1, 1)

    def body(g, _):
        for b in range(NBUF):
            i = g * NBUF + b
            bn = (b + 2) % NBUF

            pltpu.make_async_copy(tab(i), rows[b], sem_g[b]).wait()
            pltpu.make_async_copy(
                dst_hbm.at[s, 0], dstb[b], sem_di[b]).wait()

            if with_count:
                pltpu.async_copy(
                    ones_v, cnt_sp.at[dstb[b]], sem_cnt[b], add=True)
            pltpu.async_copy(
                rows[b], acc_sp.at[dstb[b]], sem_s[b], add=True)

            @pl.when(i >= 2)
            def _():
                pltpu.make_async_copy(
                    rows[bn], acc_sp.at[dstb[bn]], sem_s[bn]).wait()
                if with_count:
                    pltpu.make_async_copy(
                        ones_v, cnt_sp.at[dstb[bn]], sem_cnt[bn]).wait()

            @pl.when(i + 2 < CHUNKS)
            def _():
                pltpu.async_copy(tab(i + 2), rows[bn], sem_g[bn])
                dst_load(i + 2, bn)
        return 0

    lax.fori_loop(0, CHUNKS // NBUF, body, 0)

    for i in (CHUNKS - 2, CHUNKS - 1):
        b = i % NBUF
        pltpu.make_async_copy(
            rows[b], acc_sp.at[dstb[b]], sem_s[b]).wait()
        if with_count:
            pltpu.make_async_copy(
                ones_v, cnt_sp.at[dstb[b]], sem_cnt[b]).wait()


def _finalize(s, c, acc_sp, cnt_sp, accbuf_v, cntbuf_v, dst_hbm, col_slice):
    for b in range(NB):
        base = s * ROWS_PT + b * RB
        pltpu.sync_copy(acc_sp.at[pl.ds(base, RB)], accbuf_v)
        pltpu.sync_copy(cnt_sp.at[pl.ds(base, RB)], cntbuf_v)

        def body(r, _):
            q = 1.0 / jnp.maximum(cntbuf_v[r, pl.ds(0, 16)], 1.0)
            for k in range(DH // 16):
                accbuf_v[r, pl.ds(k * 16, 16)] = (
                    accbuf_v[r, pl.ds(k * 16, 16)] * q)
            return 0

        lax.fori_loop(0, RB, body, 0)
        if not col_slice:
            pltpu.sync_copy(accbuf_v, dst_hbm.at[pl.ds(c * NP + base, RB)])
        else:
            full_blocks = (N - s * ROWS_PT) // RB

            @pl.when(b < full_blocks)
            def _():
                pltpu.sync_copy(
                    accbuf_v, dst_hbm.at[pl.ds(base, RB), pl.ds(c * DH, DH)])

            if b == (N % ROWS_PT) // RB:
                tail = N % RB

                @pl.when(s == NS - 1)
                def _():
                    pltpu.sync_copy(
                        accbuf_v.at[pl.ds(0, tail)],
                        dst_hbm.at[pl.ds(N - tail, tail), pl.ds(c * DH, DH)])


def _gcn_body(x_hbm, src_hbm, dst_hbm, ones_hbm, zeros_hbm, h1_hbm, out_hbm,
              src_v, dstb0, dstb1, dstb2, dstb3,
              rows0_v, rows1_v, rows2_v, rows3_v, ones_v,
              accbuf_v, cntbuf_v, acc_sp, cnt_sp, *sems):
    s = lax.axis_index("s")
    c = lax.axis_index("c")
    rows = (rows0_v, rows1_v, rows2_v, rows3_v)
    dstb = (dstb0, dstb1, dstb2, dstb3)
    sem_g = sems[0:4]
    sem_di = sems[4:8]
    sem_s = sems[8:12]
    sem_cnt = sems[12:16]

    pltpu.sync_copy(src_hbm.at[c, s], src_v)

    pltpu.sync_copy(ones_hbm, ones_v)
    pltpu.sync_copy(zeros_hbm, cntbuf_v)

    _zero_2d(accbuf_v, RB, DH)
    for b in range(NB):
        pltpu.sync_copy(accbuf_v, acc_sp.at[pl.ds(s * ROWS_PT + b * RB, RB)])
        pltpu.sync_copy(cntbuf_v, cnt_sp.at[pl.ds(s * ROWS_PT + b * RB, RB)])
    plsc.subcore_barrier()

    _edge_loop(x_hbm, src_v, dst_hbm, s, dstb, rows, ones_v, acc_sp,
               cnt_sp, sem_g, sem_di, sem_s, sem_cnt, True)
    plsc.subcore_barrier()

    _finalize(s, c, acc_sp, cnt_sp, accbuf_v, cntbuf_v, h1_hbm, False)
    _zero_2d(accbuf_v, RB, DH)
    for b in range(NB):
        pltpu.sync_copy(accbuf_v, acc_sp.at[pl.ds(s * ROWS_PT + b * RB, RB)])
    plsc.subcore_barrier()

    _edge_loop(h1_hbm, src_v, dst_hbm, s, dstb, rows, ones_v, acc_sp,
               cnt_sp, sem_g, sem_di, sem_s, sem_cnt, False)
    plsc.subcore_barrier()

    _finalize(s, c, acc_sp, cnt_sp, accbuf_v, cntbuf_v, out_hbm, True)


@jax.jit
def _gcn(x2, src4, dst3, ones8, zeros8):
    f32 = jnp.float32
    kern = pl.kernel(
        _gcn_body,
        mesh=plsc.VectorSubcoreMesh(core_axis_name="c", subcore_axis_name="s"),
        out_type=[
            jax.ShapeDtypeStruct((2 * NP, DH), f32),
            jax.ShapeDtypeStruct((N, D), f32),
        ],
        scratch_types=[
            pltpu.VMEM((CHUNKS, C), jnp.int32),
            pltpu.VMEM((C,), jnp.int32),
            pltpu.VMEM((C,), jnp.int32),
            pltpu.VMEM((C,), jnp.int32),
            pltpu.VMEM((C,), jnp.int32),
            pltpu.VMEM((C, DH), f32),
            pltpu.VMEM((C, DH), f32),
            pltpu.VMEM((C, DH), f32),
            pltpu.VMEM((C, DH), f32),
            pltpu.VMEM((C, CW), f32),
            pltpu.VMEM((RB, DH), f32),
            pltpu.VMEM((RB, CW), f32),
            pltpu.VMEM_SHARED((NP, DH), f32),
            pltpu.VMEM_SHARED((NP, CW), f32),
        ] + [pltpu.SemaphoreType.DMA] * 16,
        compiler_params=pltpu.CompilerParams(use_tc_tiling_on_sc=False),
    )
    _, out2 = kern(x2, src4, dst3, ones8, zeros8)
    return out2


def kernel(x, edge_index):
    gap = jnp.zeros((NP - N, DH), jnp.float32)
    x2 = jnp.concatenate([x[:, :DH], gap, x[:, DH:]])
    src3 = edge_index[0].reshape(NS, CHUNKS, C)
    src4 = jnp.stack([src3, src3 + NP])
    dst3 = edge_index[1].reshape(NS, CHUNKS, C)
    ones8 = jnp.ones((C, CW), jnp.float32)
    zeros8 = jnp.zeros((RB, CW), jnp.float32)
    x2, src4, dst3 = lax.optimization_barrier((x2, src4, dst3))
    out2 = _gcn(x2, src4, dst3, ones8, zeros8)
    out2 = lax.optimization_barrier(out2)
    return out2[:N]


# speedup vs baseline: 12.2268x; 1.0282x over previous
import jax
import jax.numpy as jnp
from jax import lax
from jax.experimental import pallas as pl
from jax.experimental.pallas import tpu as pltpu
from jax.experimental.pallas import tpu_sc as plsc

N = 10000
NP = 10240
D = 128
E = 320000
DH = D // 2
NS = 16
C = 125
EPT = E // NS
CHUNKS = EPT // C
CW = 16
NBUF = 4
ROWS_PT = NP // NS
RB = 128
NB = ROWS_PT // RB


def _zero_2d(ref, nrows, ncols):
    zeros = jnp.zeros((16,), jnp.float32)

    def body(r, _):
        for k in range(ncols // 16):
            ref[r, pl.ds(k * 16, 16)] = zeros
        return 0

    lax.fori_loop(0, nrows, body, 0)


def _edge_loop(table_hbm, c, src_v, dst_hbm, s, dstb, rows, ones_v, acc_sp,
               cnt_sp, sem_g, sem_di, sem_s, sem_cnt, with_count):
    def tab(i):
        return table_hbm.at[c].at[src_v.at[i]]

    def dst_load(i, j):
        pltpu.async_copy(dst_hbm.at[s, i], dstb[j], sem_di[j])

    pltpu.async_copy(tab(0), rows[0], sem_g[0])
    pltpu.async_copy(tab(1), rows[1], sem_g[1])
    dst_load(0, 0)
    dst_load(1, 1)

    def body(g, _):
        for b in range(NBUF):
            i = g * NBUF + b
            bn = (b + 2) % NBUF

            pltpu.make_async_copy(tab(i), rows[b], sem_g[b]).wait()
            pltpu.make_async_copy(
                dst_hbm.at[s, 0], dstb[b], sem_di[b]).wait()

            if with_count:
                pltpu.async_copy(
                    ones_v, cnt_sp.at[dstb[b]], sem_cnt[b], add=True)
            pltpu.async_copy(
                rows[b], acc_sp.at[dstb[b]], sem_s[b], add=True)

            @pl.when(i >= 2)
            def _():
                pltpu.make_async_copy(
                    rows[bn], acc_sp.at[dstb[bn]], sem_s[bn]).wait()
                if with_count:
                    pltpu.make_async_copy(
                        ones_v, cnt_sp.at[dstb[bn]], sem_cnt[bn]).wait()

            @pl.when(i + 2 < CHUNKS)
            def _():
                pltpu.async_copy(tab(i + 2), rows[bn], sem_g[bn])
                dst_load(i + 2, bn)
        return 0

    lax.fori_loop(0, CHUNKS // NBUF, body, 0)

    for i in (CHUNKS - 2, CHUNKS - 1):
        b = i % NBUF
        pltpu.make_async_copy(
            rows[b], acc_sp.at[dstb[b]], sem_s[b]).wait()
        if with_count:
            pltpu.make_async_copy(
                ones_v, cnt_sp.at[dstb[b]], sem_cnt[b]).wait()


def _finalize(s, c, acc_sp, cnt_sp, accbuf_v, cntbuf_v, dst_hbm, col_slice):
    for b in range(NB):
        base = s * ROWS_PT + b * RB
        pltpu.sync_copy(acc_sp.at[pl.ds(base, RB)], accbuf_v)
        pltpu.sync_copy(cnt_sp.at[pl.ds(base, RB)], cntbuf_v)

        def body(r, _):
            q = 1.0 / jnp.maximum(cntbuf_v[r, pl.ds(0, 16)], 1.0)
            for k in range(DH // 16):
                accbuf_v[r, pl.ds(k * 16, 16)] = (
                    accbuf_v[r, pl.ds(k * 16, 16)] * q)
            return 0

        lax.fori_loop(0, RB, body, 0)
        if not col_slice:
            pltpu.sync_copy(accbuf_v, dst_hbm.at[c].at[pl.ds(base, RB)])
        else:
            full_blocks = (N - s * ROWS_PT) // RB

            @pl.when(b < full_blocks)
            def _():
                pltpu.sync_copy(
                    accbuf_v, dst_hbm.at[pl.ds(base, RB), pl.ds(c * DH, DH)])

            if b == (N % ROWS_PT) // RB:
                tail = N % RB

                @pl.when(s == NS - 1)
                def _():
                    pltpu.sync_copy(
                        accbuf_v.at[pl.ds(0, tail)],
                        dst_hbm.at[pl.ds(N - tail, tail), pl.ds(c * DH, DH)])


def _gcn_body(x_hbm, src_hbm, dst_hbm, ones_hbm, zeros_hbm, h1_hbm, out_hbm,
              src_v, dstb0, dstb1, dstb2, dstb3,
              rows0_v, rows1_v, rows2_v, rows3_v, ones_v,
              accbuf_v, cntbuf_v, acc_sp, cnt_sp, *sems):
    s = lax.axis_index("s")
    c = lax.axis_index("c")
    rows = (rows0_v, rows1_v, rows2_v, rows3_v)
    dstb = (dstb0, dstb1, dstb2, dstb3)
    sem_g = sems[0:4]
    sem_di = sems[4:8]
    sem_s = sems[8:12]
    sem_cnt = sems[12:16]

    pltpu.sync_copy(src_hbm.at[s], src_v)

    pltpu.sync_copy(ones_hbm, ones_v)
    pltpu.sync_copy(zeros_hbm, cntbuf_v)

    _zero_2d(accbuf_v, RB, DH)
    for b in range(NB):
        pltpu.sync_copy(accbuf_v, acc_sp.at[pl.ds(s * ROWS_PT + b * RB, RB)])
        pltpu.sync_copy(cntbuf_v, cnt_sp.at[pl.ds(s * ROWS_PT + b * RB, RB)])
    plsc.subcore_barrier()

    _edge_loop(x_hbm, c, src_v, dst_hbm, s, dstb, rows, ones_v, acc_sp,
               cnt_sp, sem_g, sem_di, sem_s, sem_cnt, True)
    plsc.subcore_barrier()

    _finalize(s, c, acc_sp, cnt_sp, accbuf_v, cntbuf_v, h1_hbm, False)
    _zero_2d(accbuf_v, RB, DH)
    for b in range(NB):
        pltpu.sync_copy(accbuf_v, acc_sp.at[pl.ds(s * ROWS_PT + b * RB, RB)])
    plsc.subcore_barrier()

    _edge_loop(h1_hbm, c, src_v, dst_hbm, s, dstb, rows, ones_v, acc_sp,
               cnt_sp, sem_g, sem_di, sem_s, sem_cnt, False)
    plsc.subcore_barrier()

    _finalize(s, c, acc_sp, cnt_sp, accbuf_v, cntbuf_v, out_hbm, True)


@jax.jit
def _gcn(x2, src3, dst3, ones8, zeros8):
    f32 = jnp.float32
    kern = pl.kernel(
        _gcn_body,
        mesh=plsc.VectorSubcoreMesh(core_axis_name="c", subcore_axis_name="s"),
        out_type=[
            jax.ShapeDtypeStruct((2, NP, DH), f32),
            jax.ShapeDtypeStruct((N, D), f32),
        ],
        scratch_types=[
            pltpu.VMEM((CHUNKS, C), jnp.int32),
            pltpu.VMEM((C,), jnp.int32),
            pltpu.VMEM((C,), jnp.int32),
            pltpu.VMEM((C,), jnp.int32),
            pltpu.VMEM((C,), jnp.int32),
            pltpu.VMEM((C, DH), f32),
            pltpu.VMEM((C, DH), f32),
            pltpu.VMEM((C, DH), f32),
            pltpu.VMEM((C, DH), f32),
            pltpu.VMEM((C, CW), f32),
            pltpu.VMEM((RB, DH), f32),
            pltpu.VMEM((RB, CW), f32),
            pltpu.VMEM_SHARED((NP, DH), f32),
            pltpu.VMEM_SHARED((NP, CW), f32),
        ] + [pltpu.SemaphoreType.DMA] * 16,
        compiler_params=pltpu.CompilerParams(use_tc_tiling_on_sc=False),
    )
    _, out2 = kern(x2, src3, dst3, ones8, zeros8)
    return out2


def kernel(x, edge_index):
    gap = jnp.zeros((NP - N, DH), jnp.float32)
    x2 = jnp.concatenate(
        [x[:, :DH], gap, x[:, DH:], gap]).reshape(2, NP, DH)
    src3 = edge_index[0].reshape(NS, CHUNKS, C)
    dst3 = edge_index[1].reshape(NS, CHUNKS, C)
    ones8 = jnp.ones((C, CW), jnp.float32)
    zeros8 = jnp.zeros((RB, CW), jnp.float32)
    return _gcn(x2, src3, dst3, ones8, zeros8)


# speedup vs baseline: 13.1585x; 1.0762x over previous
import jax
import jax.numpy as jnp
from jax import lax
from jax.experimental import pallas as pl
from jax.experimental.pallas import tpu as pltpu
from jax.experimental.pallas import tpu_sc as plsc

N = 10000
NP = 10240
D = 128
E = 320000
DH = D // 2
NS = 16
C = 125
EPT = E // NS
CHUNKS = EPT // C
CW = 16
NBUF = 4
ROWS_PT = NP // NS
RB = 128
NB = ROWS_PT // RB


def _zero_2d(ref, nrows, ncols):
    zeros = jnp.zeros((16,), jnp.float32)

    def body(r, _):
        for k in range(ncols // 16):
            ref[r, pl.ds(k * 16, 16)] = zeros
        return 0

    lax.fori_loop(0, nrows, body, 0)


def _edge_loop(table_hbm, c, src_v, dst_hbm, s, dstb, rows, ones_v, acc_sp,
               cnt_sp, sem_g, sem_di, sem_s, sem_cnt, with_count):
    def tab(i):
        return table_hbm.at[c].at[src_v.at[i]]

    def dst_load(i, j):
        pltpu.async_copy(dst_hbm.at[s, i], dstb[j], sem_di[j])

    pltpu.async_copy(tab(0), rows[0], sem_g[0])
    pltpu.async_copy(tab(1), rows[1], sem_g[1])
    dst_load(0, 0)
    dst_load(1, 1)

    def body(g, _):
        for b in range(NBUF):
            i = g * NBUF + b
            bn = (b + 2) % NBUF

            pltpu.make_async_copy(tab(i), rows[b], sem_g[b]).wait()
            pltpu.make_async_copy(
                dst_hbm.at[s, 0], dstb[b], sem_di[b]).wait()

            if with_count:
                pltpu.async_copy(
                    ones_v, cnt_sp.at[dstb[b]], sem_cnt[b], add=True)
            pltpu.async_copy(
                rows[b], acc_sp.at[dstb[b]], sem_s[b], add=True)

            @pl.when(i >= 2)
            def _():
                pltpu.make_async_copy(
                    rows[bn], acc_sp.at[dstb[bn]], sem_s[bn]).wait()
                if with_count:
                    pltpu.make_async_copy(
                        ones_v, cnt_sp.at[dstb[bn]], sem_cnt[bn]).wait()

            @pl.when(i + 2 < CHUNKS)
            def _():
                pltpu.async_copy(tab(i + 2), rows[bn], sem_g[bn])
                dst_load(i + 2, bn)
        return 0

    lax.fori_loop(0, CHUNKS // NBUF, body, 0)

    for i in (CHUNKS - 2, CHUNKS - 1):
        b = i % NBUF
        pltpu.make_async_copy(
            rows[b], acc_sp.at[dstb[b]], sem_s[b]).wait()
        if with_count:
            pltpu.make_async_copy(
                ones_v, cnt_sp.at[dstb[b]], sem_cnt[b]).wait()


def _finalize(s, c, acc_sp, cnt_sp, accbuf_v, cntbuf_v, dst_hbm, col_slice):
    for b in range(NB):
        base = s * ROWS_PT + b * RB
        pltpu.sync_copy(acc_sp.at[pl.ds(base, RB)], accbuf_v)
        pltpu.sync_copy(cnt_sp.at[pl.ds(base, RB)], cntbuf_v)

        def body(r, _):
            q = 1.0 / jnp.maximum(cntbuf_v[r, pl.ds(0, 16)], 1.0)
            for k in range(DH // 16):
                accbuf_v[r, pl.ds(k * 16, 16)] = (
                    accbuf_v[r, pl.ds(k * 16, 16)] * q)
            return 0

        lax.fori_loop(0, RB, body, 0)
        if not col_slice:
            pltpu.sync_copy(accbuf_v, dst_hbm.at[c].at[pl.ds(base, RB)])
        else:
            full_blocks = (N - s * ROWS_PT) // RB

            @pl.when(b < full_blocks)
            def _():
                pltpu.sync_copy(
                    accbuf_v, dst_hbm.at[pl.ds(base, RB), pl.ds(c * DH, DH)])

            if b == (N % ROWS_PT) // RB:
                tail = N % RB

                @pl.when(s == NS - 1)
                def _():
                    pltpu.sync_copy(
                        accbuf_v.at[pl.ds(0, tail)],
                        dst_hbm.at[pl.ds(N - tail, tail), pl.ds(c * DH, DH)])


def _gcn_body(x_hbm, src_hbm, dst_hbm, ones_hbm, zeros_hbm, xh_hbm, out_hbm,
              src_v, dstb0, dstb1, dstb2, dstb3,
              rows0_v, rows1_v, rows2_v, rows3_v, ones_v,
              accbuf_v, cntbuf_v, acc_sp, cnt_sp, *sems):
    s = lax.axis_index("s")
    c = lax.axis_index("c")
    rows = (rows0_v, rows1_v, rows2_v, rows3_v)
    dstb = (dstb0, dstb1, dstb2, dstb3)
    sem_g = sems[0:4]
    sem_di = sems[4:8]
    sem_s = sems[8:12]
    sem_cnt = sems[12:16]

    pltpu.sync_copy(src_hbm.at[s], src_v)

    full_blocks = (N - s * ROWS_PT) // RB
    for b in range(NB):
        base = s * ROWS_PT + b * RB

        @pl.when(b < full_blocks)
        def _():
            pltpu.sync_copy(
                x_hbm.at[pl.ds(base, RB), pl.ds(c * DH, DH)], accbuf_v)
            pltpu.sync_copy(accbuf_v, xh_hbm.at[c].at[pl.ds(base, RB)])

        if b == (N % ROWS_PT) // RB:
            tail = N % RB

            @pl.when(s == NS - 1)
            def _():
                pltpu.sync_copy(
                    x_hbm.at[pl.ds(N - tail, tail), pl.ds(c * DH, DH)],
                    accbuf_v.at[pl.ds(0, tail)])
                pltpu.sync_copy(accbuf_v.at[pl.ds(0, tail)],
                                xh_hbm.at[c].at[pl.ds(N - tail, tail)])

    pltpu.sync_copy(ones_hbm, ones_v)
    pltpu.sync_copy(zeros_hbm, cntbuf_v)

    _zero_2d(accbuf_v, RB, DH)
    for b in range(NB):
        pltpu.sync_copy(accbuf_v, acc_sp.at[pl.ds(s * ROWS_PT + b * RB, RB)])
        pltpu.sync_copy(cntbuf_v, cnt_sp.at[pl.ds(s * ROWS_PT + b * RB, RB)])
    plsc.subcore_barrier()

    _edge_loop(xh_hbm, c, src_v, dst_hbm, s, dstb, rows, ones_v, acc_sp,
               cnt_sp, sem_g, sem_di, sem_s, sem_cnt, True)
    plsc.subcore_barrier()

    _finalize(s, c, acc_sp, cnt_sp, accbuf_v, cntbuf_v, xh_hbm, False)
    _zero_2d(accbuf_v, RB, DH)
    for b in range(NB):
        pltpu.sync_copy(accbuf_v, acc_sp.at[pl.ds(s * ROWS_PT + b * RB, RB)])
    plsc.subcore_barrier()

    _edge_loop(xh_hbm, c, src_v, dst_hbm, s, dstb, rows, ones_v, acc_sp,
               cnt_sp, sem_g, sem_di, sem_s, sem_cnt, False)
    plsc.subcore_barrier()

    _finalize(s, c, acc_sp, cnt_sp, accbuf_v, cntbuf_v, out_hbm, True)


@jax.jit
def _gcn(x, src3, dst3, ones8, zeros8):
    f32 = jnp.float32
    kern = pl.kernel(
        _gcn_body,
        mesh=plsc.VectorSubcoreMesh(core_axis_name="c", subcore_axis_name="s"),
        out_type=[
            jax.ShapeDtypeStruct((2, NP, DH), f32),
            jax.ShapeDtypeStruct((N, D), f32),
        ],
        scratch_types=[
            pltpu.VMEM((CHUNKS, C), jnp.int32),
            pltpu.VMEM((C,), jnp.int32),
            pltpu.VMEM((C,), jnp.int32),
            pltpu.VMEM((C,), jnp.int32),
            pltpu.VMEM((C,), jnp.int32),
            pltpu.VMEM((C, DH), f32),
            pltpu.VMEM((C, DH), f32),
            pltpu.VMEM((C, DH), f32),
            pltpu.VMEM((C, DH), f32),
            pltpu.VMEM((C, CW), f32),
            pltpu.VMEM((RB, DH), f32),
            pltpu.VMEM((RB, CW), f32),
            pltpu.VMEM_SHARED((NP, DH), f32),
            pltpu.VMEM_SHARED((NP, CW), f32),
        ] + [pltpu.SemaphoreType.DMA] * 16,
        compiler_params=pltpu.CompilerParams(use_tc_tiling_on_sc=False),
    )
    _, out2 = kern(x, src3, dst3, ones8, zeros8)
    return out2


def kernel(x, edge_index):
    src3 = edge_index[0].reshape(NS, CHUNKS, C)
    dst3 = edge_index[1].reshape(NS, CHUNKS, C)
    ones8 = jnp.ones((C, CW), jnp.float32)
    zeros8 = jnp.zeros((RB, CW), jnp.float32)
    return _gcn(x, src3, dst3, ones8, zeros8)
